# baseline (speedup 1.0000x reference)
import jax
import jax.numpy as jnp
from jax import lax
from jax.experimental import pallas as pl
from jax.experimental.pallas import tpu as pltpu
from jax.experimental.pallas import tpu_sc as plsc

_N = 10000
_NHID = 16
_NSC = 1536
_NW = 32
_RW = _NSC // _NW
_NP = _RW // 2
_KC = 2000
_NKC = _N // _KC
_TI = _KC // 16


def _sc_mm(adj1f, adj2f, xw1t, xw2t, e1, e2,
           xwt_buf, abufA, abufB, elan, semA, semB):
    c = lax.axis_index("c")
    s = lax.axis_index("s")
    wid = s * 2 + c
    base = (_N - _NSC) + wid * _RW

    zero16 = jnp.zeros((16,), jnp.float32)

    def issue(adjf, row, k0, buf, sem):
        pltpu.async_copy(
            adjf.at[pl.ds(row * _N + k0, _KC)], buf.at[pl.ds(0, _KC)], sem)
        pltpu.async_copy(
            adjf.at[pl.ds((row + 1) * _N + k0, _KC)],
            buf.at[pl.ds(_KC, _KC)], sem)

    def wait2(adjf, buf, sem):
        pltpu.make_async_copy(
            adjf.at[pl.ds(0, _KC)], buf.at[pl.ds(0, _KC)], sem).wait()
        pltpu.make_async_copy(
            adjf.at[pl.ds(0, _KC)], buf.at[pl.ds(_KC, _KC)], sem).wait()

    def do_mat(adjf, xwt, eout):
        def zbody(r, _):
            for j in range(16):
                elan[r, pl.ds(j * 16, 16)] = zero16
            return 0
        lax.fori_loop(0, _RW, zbody, 0)

        def kbody(kc, _):
            k0 = kc * _KC
            pltpu.sync_copy(xwt.at[kc], xwt_buf)
            issue(adjf, base, k0, abufA, semA)

            def inner(buf, accs):
                def ibody(t, accs):
                    a0 = buf[pl.ds(t * 16, 16)]
                    a1 = buf[pl.ds(_KC + t * 16, 16)]
                    out = []
                    for j in range(16):
                        w = xwt_buf[j, pl.ds(t * 16, 16)]
                        out.append(accs[2 * j] + a0 * w)
                        out.append(accs[2 * j + 1] + a1 * w)
                    return tuple(out)
                return lax.fori_loop(0, _TI, ibody, accs)

            def flush(p, accs):
                rl = 2 * p
                for j in range(16):
                    ds = pl.ds(j * 16, 16)
                    elan[rl, ds] = elan[rl, ds] + accs[2 * j]
                    elan[rl + 1, ds] = elan[rl + 1, ds] + accs[2 * j + 1]

            init = tuple(zero16 for _ in range(32))

            def pbody(i, _):
                pA = 2 * i
                pB = 2 * i + 1
                issue(adjf, base + 2 * pB, k0, abufB, semB)
                wait2(adjf, abufA, semA)
                accs = inner(abufA, init)
                flush(pA, accs)

                @pl.when(pA + 2 < _NP)
                def _():
                    issue(adjf, base + 2 * (pA + 2), k0, abufA, semA)

                wait2(adjf, abufB, semB)
                accs = inner(abufB, init)
                flush(pB, accs)
                return 0

            lax.fori_loop(0, _NP // 2, pbody, 0)
            return 0

        lax.fori_loop(0, _NKC, kbody, 0)

        pltpu.sync_copy(elan, eout.at[pl.ds(wid * _RW, _RW), :])

    do_mat(adj1f, xw1t, e1)
    do_mat(adj2f, xw2t, e2)


def _sc_mm_call(adj1f, adj2f, xw1t3, xw2t3):
    mesh = plsc.VectorSubcoreMesh(core_axis_name="c", subcore_axis_name="s")
    f = pl.kernel(
        _sc_mm,
        mesh=mesh,
        out_type=[
            jax.ShapeDtypeStruct((_NSC, 256), jnp.float32),
            jax.ShapeDtypeStruct((_NSC, 256), jnp.float32),
        ],
        scratch_types=[
            pltpu.VMEM((16, _KC), jnp.float32),
            pltpu.VMEM((2 * _KC,), jnp.float32),
            pltpu.VMEM((2 * _KC,), jnp.float32),
            pltpu.VMEM((_RW, 256), jnp.float32),
            pltpu.SemaphoreType.DMA,
            pltpu.SemaphoreType.DMA,
        ],
    )
    return f(adj1f, adj2f, xw1t3, xw2t3)


def kernel(x, adj1, adj2, W1, b1, W2, b2, Wa, cluster):
    xw1t3 = adj1[:16, :].reshape(16, _NKC, _KC).transpose(1, 0, 2)
    xw2t3 = adj2[:16, :].reshape(16, _NKC, _KC).transpose(1, 0, 2)
    e1t, e2t = _sc_mm_call(adj1.reshape(-1), adj2.reshape(-1), xw1t3, xw2t3)
    xo = jnp.zeros((_N, _NHID), jnp.float32).at[_N - _NSC:, :].set(e1t[:, :16])
    q = jnp.zeros((_N, 10), jnp.float32).at[_N - _NSC:, :].set(e2t[:, :10])
    return (xo, q)


# speedup vs baseline: 4.5713x; 4.5713x over previous
import jax
import jax.numpy as jnp
from jax.experimental import pallas as pl
from jax.experimental.pallas import tpu as pltpu

_N = 10000
_NFEAT = 128
_NHID = 16
_NCLASS = 10
_BLK = 200
_ALPHA = 0.2
_POW = 0.72


def _fused(x_ref, adj1_ref, adj2_ref, w1_ref, w2_ref, b1_ref, b2_ref,
           wa_ref, ct_ref, xo_ref, q_ref, xw1_ref, xw2_ref):
    i = pl.program_id(0)

    @pl.when(i == 0)
    def _():
        xb = x_ref[...].astype(jnp.bfloat16)
        xw1 = jnp.dot(xb, w1_ref[...].astype(jnp.bfloat16),
                      preferred_element_type=jnp.float32)
        xw2 = jnp.dot(xb, w2_ref[...].astype(jnp.bfloat16),
                      preferred_element_type=jnp.float32)
        hi1 = xw1.astype(jnp.bfloat16)
        hi2 = xw2.astype(jnp.bfloat16)
        lo1 = (xw1 - hi1.astype(jnp.float32)).astype(jnp.bfloat16)
        lo2 = (xw2 - hi2.astype(jnp.float32)).astype(jnp.bfloat16)
        xw1_ref[...] = jnp.concatenate([hi1, lo1], axis=1)
        xw2_ref[...] = jnp.concatenate([hi2, lo2], axis=1)

    a1 = adj1_ref[...].astype(jnp.bfloat16)
    a2 = adj2_ref[...].astype(jnp.bfloat16)
    ee1 = jnp.dot(a1, xw1_ref[...], preferred_element_type=jnp.float32)
    ee2 = jnp.dot(a2, xw2_ref[...], preferred_element_type=jnp.float32)
    e1 = ee1[:, :_NHID] + ee1[:, _NHID:] + b1_ref[...]
    e2 = ee2[:, :_NHID] + ee2[:, _NHID:] + b2_ref[...]

    s1 = jnp.sum(e1 * wa_ref[...], axis=1, keepdims=True)
    s2 = jnp.sum(e2 * wa_ref[...], axis=1, keepdims=True)
    m = jnp.maximum(s1, s2)
    p1 = jnp.exp(s1 - m)
    p2 = jnp.exp(s2 - m)
    xo = (p1 * e1 + p2 * e2) / (p1 + p2)
    xo_ref[...] = xo

    ct = ct_ref[...]
    csq = jnp.sum(ct * ct, axis=0, keepdims=True)
    cross = jnp.dot(xo, ct, preferred_element_type=jnp.float32)
    dist = jnp.sum(xo * xo, axis=1, keepdims=True) - 2.0 * cross + csq
    p = 1.0 / (1.0 + dist * (1.0 / _ALPHA))
    qu = jnp.exp(_POW * jnp.log(p))
    q_ref[...] = qu / jnp.sum(qu, axis=1, keepdims=True)


def kernel(x, adj1, adj2, W1, b1, W2, b2, Wa, cluster):
    b1r = b1.reshape(1, _NHID)
    b2r = b2.reshape(1, _NHID)
    war = Wa.reshape(1, _NHID)
    ct = cluster.T

    grid = (_N // _BLK,)
    xo, q = pl.pallas_call(
        _fused,
        grid=grid,
        in_specs=[
            pl.BlockSpec((_N, _NFEAT), lambda i: (0, 0)),
            pl.BlockSpec((_BLK, _N), lambda i: (i, 0)),
            pl.BlockSpec((_BLK, _N), lambda i: (i, 0)),
            pl.BlockSpec((_NFEAT, _NHID), lambda i: (0, 0)),
            pl.BlockSpec((_NFEAT, _NHID), lambda i: (0, 0)),
            pl.BlockSpec((1, _NHID), lambda i: (0, 0)),
            pl.BlockSpec((1, _NHID), lambda i: (0, 0)),
            pl.BlockSpec((1, _NHID), lambda i: (0, 0)),
            pl.BlockSpec((_NHID, _NCLASS), lambda i: (0, 0)),
        ],
        out_specs=[
            pl.BlockSpec((_BLK, _NHID), lambda i: (i, 0)),
            pl.BlockSpec((_BLK, _NCLASS), lambda i: (i, 0)),
        ],
        out_shape=[
            jax.ShapeDtypeStruct((_N, _NHID), jnp.float32),
            jax.ShapeDtypeStruct((_N, _NCLASS), jnp.float32),
        ],
        scratch_shapes=[
            pltpu.VMEM((_N, 2 * _NHID), jnp.bfloat16),
            pltpu.VMEM((_N, 2 * _NHID), jnp.bfloat16),
        ],
    )(x, adj1, adj2, W1, W2, b1r, b2r, war, ct)
    return (xo, q)
